# baseline (speedup 1.0000x reference)
import functools

import jax
import jax.numpy as jnp
from jax.experimental import pallas as pl
from jax.experimental.pallas import tpu as pltpu

_BN_EPS = 1e-5


def _round_up(a, m):
    return (a + m - 1) // m * m


def _compiler_params():
    return pltpu.CompilerParams(
        dimension_semantics=("parallel",),
        vmem_limit_bytes=56 * 1024 * 1024,
    )


def _full(shape):
    nd = len(shape)
    return pl.BlockSpec(shape, lambda q, _nd=nd: (0,) * _nd)


def _scale_shift(st_ref, g_ref, b_ref, m_count):
    total = jnp.sum(st_ref[...], axis=0)
    mean = total[0:1, :] / m_count
    var = jnp.maximum(total[1:2, :] / m_count - mean * mean, 0.0)
    inv = jax.lax.rsqrt(var + _BN_EPS)
    sc = g_ref[...] * inv
    sh = b_ref[...] - mean * sc
    return sc, sh


def _embed_plane(src):
    p = jnp.pad(src, ((1, 1), (1, 1), (0, 0)))
    return p.reshape(-1, src.shape[-1])


def _repack_body(xa_ref, xb_ref, pee, peo, poe, poo, *, nine):
    q = pl.program_id(0)
    j = jnp.maximum(q - 1, 0) % nine
    outs = (pee, peo, poe, poo)
    for pd in (0, 1):
        planes = []
        for ref in (xa_ref, xb_ref):
            p = ref[0, pd]
            planes.append(p.reshape(p.shape[0] // 2, 2, *p.shape[1:]))
        for phh in (0, 1):
            halves = []
            for rel in (0, 1):
                src = planes[rel][:, phh, :, :]
                blk = _embed_plane(src)
                dead = (j == 0) if rel == 0 else (j == nine - 1)
                dead = jnp.logical_or(dead, q == 0)
                halves.append(jnp.where(dead, 0.0, blk))
            out = jnp.concatenate(halves, axis=0).astype(jnp.bfloat16)
            outs[2 * pd + phh][...] = out


def _down_body(pee, peo, poe, poo, w_ref, b_ref, mask_ref, y_ref, s_ref,
               *, R, RB, DH, taps, T):
    q = pl.program_id(0)
    ph = (pee, peo, poe, poo)
    C = w_ref.shape[2]
    qc = jnp.clip(q, 1, T)
    ws = (qc - 1) * R + (RB - DH)
    wide = []
    for p in ph:
        w = p[pl.ds(ws, DH + R), :]
        wide.append(jnp.concatenate([w[:-1], w[1:]], axis=1))
    acc = jnp.zeros((R, C), jnp.float32)
    for i in range(0, len(taps), 2):
        p1, o1 = taps[i]
        a = wide[p1][o1 + DH:o1 + DH + R, :]
        acc = acc + jnp.dot(a, w_ref[i // 2],
                            preferred_element_type=jnp.float32)
    active = jnp.logical_and(q >= 1, q <= T)
    y = jnp.where(active, acc + b_ref[...], 0.0)
    y_ref[...] = y.astype(y_ref.dtype)
    ym = y * mask_ref[...]
    s_ref[0, 0:1, :] = jnp.sum(ym, axis=0, keepdims=True)
    s_ref[0, 1:2, :] = jnp.sum(ym * ym, axis=0, keepdims=True)


def _conv_body(yext_ref, st_ref, g_ref, b_ref, w_ref, wl_ref, bias_ref,
               maskext_ref, o_ref, s_ref,
               *, R, HALO, offs, T, m_count):
    q = pl.program_id(0)
    C = w_ref.shape[2]
    WIN = R + 2 * HALO
    sc, sh = _scale_shift(st_ref, g_ref, b_ref, m_count)
    qc = jnp.clip(q, 1, T)
    ws = (qc - 1) * R + (R - HALO)
    ywin = yext_ref[pl.ds(ws, WIN), :]
    mwin = maskext_ref[pl.ds(ws, WIN), :]
    z = (jnp.maximum(ywin * sc + sh, 0.0) * mwin).astype(jnp.bfloat16)
    acc = jnp.zeros((R, C), jnp.float32)
    for i in range(len(offs) // 2):
        o1, o2 = offs[2 * i], offs[2 * i + 1]
        a = jnp.concatenate(
            [z[o1 + HALO:o1 + HALO + R, :],
             z[o2 + HALO:o2 + HALO + R, :]], axis=1)
        acc = acc + jnp.dot(a, w_ref[i],
                            preferred_element_type=jnp.float32)
    if len(offs) % 2:
        o = offs[-1]
        acc = acc + jnp.dot(z[o + HALO:o + HALO + R, :], wl_ref[...],
                            preferred_element_type=jnp.float32)
    active = jnp.logical_and(q >= 1, q <= T)
    y = jnp.where(active, acc + bias_ref[...], 0.0)
    o_ref[...] = y.astype(o_ref.dtype)
    ym = y * maskext_ref[pl.ds(q * R, R), :]
    s_ref[0, 0:1, :] = jnp.sum(ym, axis=0, keepdims=True)
    s_ref[0, 1:2, :] = jnp.sum(ym * ym, axis=0, keepdims=True)


def _final_body(o1_ref, y_ref, st_ref, g_ref, b_ref, o_ref, *, m_count):
    sc, sh = _scale_shift(st_ref, g_ref, b_ref, m_count)
    o_ref[...] = jnp.maximum(o1_ref[...] + y_ref[...] * sc + sh, 0.0)


def kernel(x, down_w, down_b, bn_g, bn_b, conv0_w, conv0_b, bn0_g, bn0_b,
           conv1_w, conv1_b, bn1_g, bn1_b):
    N, Cin, D, H, W = x.shape
    Cout = down_w.shape[0]
    Do, Ho, Wo = D // 2, H // 2, W // 2
    Dp, Hp, Wp = Do + 2, Ho + 2, Wo + 2
    rows_n = Dp * Hp * Wp
    ROWS = N * rows_n
    RB = 2 * Hp * Wp
    R = 4 * RB
    T = ROWS // R
    TB = ROWS // RB
    EXT = ROWS + 2 * R
    PH_ROWS = RB + ROWS
    HALO = _round_up(Hp * Wp + Wp + 1, 8)
    M = float(N * Do * Ho * Wo)

    f32, bf16 = jnp.float32, jnp.bfloat16

    xw = jnp.transpose(x, (0, 2, 3, 4, 1)).reshape(N, D, H, Wo, 2 * Cin)

    rg = jnp.arange(EXT) - R
    rem = jnp.where(rg >= 0, rg, 0) % rows_n
    d = rem // (Hp * Wp)
    h = (rem % (Hp * Wp)) // Wp
    w_ = rem % Wp
    ok = ((rg >= 0) & (rg < ROWS)
          & (d >= 1) & (d < Dp - 1) & (h >= 1) & (h < Hp - 1)
          & (w_ >= 1) & (w_ < Wp - 1))
    mask = jnp.broadcast_to(ok.astype(bf16)[:, None], (EXT, Cout))

    wt = jnp.transpose(down_w, (1, 2, 3, 4, 0))
    zblk = jnp.zeros((Cin, Cout), f32)
    blocks, taps = [], []
    for kd in range(3):
        pd = 0 if kd == 1 else 1
        sd = -1 if kd == 0 else 0
        for kh in range(3):
            phh = 0 if kh == 1 else 1
            sh_ = -1 if kh == 0 else 0
            for g in (0, 1):
                sw = -1 if g == 0 else 0
                off = sd * (Hp * Wp) + sh_ * Wp + sw
                if g == 0:
                    blk = jnp.concatenate([zblk, wt[:, kd, kh, 0, :]], axis=0)
                else:
                    blk = jnp.concatenate(
                        [wt[:, kd, kh, 1, :], wt[:, kd, kh, 2, :]], axis=0)
                blocks.append(blk)
                taps.append((pd * 2 + phh, off))
    w18 = jnp.stack(blocks).astype(bf16)
    w9 = jnp.concatenate([w18[0::2], w18[1::2]], axis=1)

    def conv_wp(w):
        w27 = jnp.transpose(w, (2, 3, 4, 1, 0)).reshape(27, Cout, Cout)
        wp = jnp.concatenate([w27[0:26:2], w27[1:26:2]], axis=1)
        return wp.astype(bf16), w27[26].astype(bf16)

    offs27 = [dd * (Hp * Wp) + hh * Wp + ww
              for dd in (-1, 0, 1) for hh in (-1, 0, 1) for ww in (-1, 0, 1)]

    nine = TB // N
    def ima(q):
        s = jnp.maximum(q - 1, 0)
        return (s // nine, jnp.clip(2 * (s % nine) - 1, 0, Do - 1), 0, 0, 0)
    def imb(q):
        s = jnp.maximum(q - 1, 0)
        return (s // nine, jnp.clip(2 * (s % nine), 0, Do - 1), 0, 0, 0)
    phases = pl.pallas_call(
        functools.partial(_repack_body, nine=nine),
        grid=(TB + 1,),
        in_specs=[
            pl.BlockSpec((1, 2, H, Wo, Cout), ima),
            pl.BlockSpec((1, 2, H, Wo, Cout), imb),
        ],
        out_specs=[pl.BlockSpec((RB, Cout), lambda q: (q, 0))] * 4,
        out_shape=[jax.ShapeDtypeStruct((PH_ROWS, Cout), bf16)] * 4,
        compiler_params=_compiler_params(),
    )(xw, xw)

    y1_ext, st = pl.pallas_call(
        functools.partial(_down_body, R=R, RB=RB, DH=HALO, taps=taps, T=T),
        grid=(T + 2,),
        in_specs=[_full((PH_ROWS, Cout))] * 4 + [
            _full(w9.shape),
            _full((1, Cout)),
            pl.BlockSpec((R, Cout), lambda q: (q, 0)),
        ],
        out_specs=[
            pl.BlockSpec((R, Cout), lambda q: (q, 0)),
            pl.BlockSpec((1, 8, Cout), lambda q: (q, 0, 0)),
        ],
        out_shape=[
            jax.ShapeDtypeStruct((EXT, Cout), bf16),
            jax.ShapeDtypeStruct((T + 2, 8, Cout), f32),
        ],
        compiler_params=_compiler_params(),
    )(*phases, w9, down_b.reshape(1, Cout), mask)

    def conv_call(y_ext, st, g, b, wpair, bias):
        wp, wl = wpair
        return pl.pallas_call(
            functools.partial(_conv_body, R=R, HALO=HALO, offs=offs27,
                              T=T, m_count=M),
            grid=(T + 2,),
            in_specs=[
                _full((EXT, Cout)),
                _full((T + 2, 8, Cout)),
                _full((1, Cout)),
                _full((1, Cout)),
                _full(wp.shape),
                _full(wl.shape),
                _full((1, Cout)),
                _full((EXT, Cout)),
            ],
            out_specs=[
                pl.BlockSpec((R, Cout), lambda q: (q, 0)),
                pl.BlockSpec((1, 8, Cout), lambda q: (q, 0, 0)),
            ],
            out_shape=[
                jax.ShapeDtypeStruct((EXT, Cout), bf16),
                jax.ShapeDtypeStruct((T + 2, 8, Cout), f32),
            ],
            compiler_params=_compiler_params(),
        )(y_ext, st, g.reshape(1, Cout), b.reshape(1, Cout), wp, wl,
          bias.reshape(1, Cout), mask)

    y2_ext, st0 = conv_call(y1_ext, st, bn_g, bn_b,
                            conv_wp(conv0_w), conv0_b)
    y3_ext, st1 = conv_call(y2_ext, st0, bn0_g, bn0_b,
                            conv_wp(conv1_w), conv1_b)

    out_ext = pl.pallas_call(
        functools.partial(_final_body, m_count=M),
        grid=(T,),
        in_specs=[
            pl.BlockSpec((R, Cout), lambda m: (m + 1, 0)),
            pl.BlockSpec((R, Cout), lambda m: (m + 1, 0)),
            _full((T + 2, 8, Cout)),
            _full((1, Cout)),
            _full((1, Cout)),
        ],
        out_specs=pl.BlockSpec((R, Cout), lambda m: (m + 1, 0)),
        out_shape=jax.ShapeDtypeStruct((EXT, Cout), f32),
        compiler_params=_compiler_params(),
    )(y1_ext, y3_ext, st1, bn1_g.reshape(1, Cout), bn1_b.reshape(1, Cout))

    out = out_ext[R:R + ROWS].reshape(N, Dp, Hp, Wp, Cout)
    out = out[:, 1:Dp - 1, 1:Hp - 1, 1:Wp - 1, :]
    return jnp.transpose(out, (0, 4, 1, 2, 3))


# speedup vs baseline: 1.0087x; 1.0087x over previous
import functools

import jax
import jax.numpy as jnp
from jax.experimental import pallas as pl
from jax.experimental.pallas import tpu as pltpu

_BN_EPS = 1e-5


def _round_up(a, m):
    return (a + m - 1) // m * m


def _compiler_params():
    return pltpu.CompilerParams(
        dimension_semantics=("parallel",),
        vmem_limit_bytes=56 * 1024 * 1024,
    )


def _full(shape):
    nd = len(shape)
    return pl.BlockSpec(shape, lambda q, _nd=nd: (0,) * _nd)


def _scale_shift(st_ref, g_ref, b_ref, m_count):
    total = jnp.sum(st_ref[...], axis=0)
    mean = total[0:1, :] / m_count
    var = jnp.maximum(total[1:2, :] / m_count - mean * mean, 0.0)
    inv = jax.lax.rsqrt(var + _BN_EPS)
    sc = g_ref[...] * inv
    sh = b_ref[...] - mean * sc
    return sc, sh


def _embed_plane(src):
    p = jnp.pad(src, ((1, 1), (1, 1), (0, 0)))
    return p.reshape(-1, src.shape[-1])


def _repack_body(xa_ref, xb_ref, pee, peo, poe, poo, *, nine):
    q = pl.program_id(0)
    j = jnp.maximum(q - 1, 0) % nine
    outs = (pee, peo, poe, poo)
    for pd in (0, 1):
        planes = []
        for ref in (xa_ref, xb_ref):
            p = ref[0, pd]
            planes.append(p.reshape(p.shape[0] // 2, 2, *p.shape[1:]))
        for phh in (0, 1):
            halves = []
            for rel in (0, 1):
                src = planes[rel][:, phh, :, :]
                blk = _embed_plane(src)
                dead = (j == 0) if rel == 0 else (j == nine - 1)
                dead = jnp.logical_or(dead, q == 0)
                halves.append(jnp.where(dead, 0.0, blk))
            out = jnp.concatenate(halves, axis=0).astype(jnp.bfloat16)
            outs[2 * pd + phh][...] = out


def _down_body(pee, peo, poe, poo, w_ref, b_ref, mask_ref, y_ref, s_ref,
               *, R, RB, DH, taps, T):
    q = pl.program_id(0)
    ph = (pee, peo, poe, poo)
    C = w_ref.shape[2]
    qc = jnp.clip(q, 1, T)
    ws = (qc - 1) * R + (RB - DH)
    wide = []
    for p in ph:
        w = p[pl.ds(ws, DH + R), :]
        wide.append(jnp.concatenate([w[:-1], w[1:]], axis=1))
    acc = jnp.zeros((R, C), jnp.float32)
    for i in range(0, len(taps), 2):
        p1, o1 = taps[i]
        a = wide[p1][o1 + DH:o1 + DH + R, :]
        acc = acc + jnp.dot(a, w_ref[i // 2],
                            preferred_element_type=jnp.float32)
    active = jnp.logical_and(q >= 1, q <= T)
    y = jnp.where(active, acc + b_ref[...], 0.0)
    y_ref[...] = y.astype(y_ref.dtype)
    ym = y * mask_ref[...]
    s_ref[0, 0:1, :] = jnp.sum(ym, axis=0, keepdims=True)
    s_ref[0, 1:2, :] = jnp.sum(ym * ym, axis=0, keepdims=True)


def _conv_body(yext_ref, st_ref, g_ref, b_ref, w_ref, wl_ref, bias_ref,
               maskext_ref, o_ref, s_ref,
               *, R, HALO, offs, T, m_count):
    q = pl.program_id(0)
    C = w_ref.shape[2]
    WIN = R + 2 * HALO
    sc, sh = _scale_shift(st_ref, g_ref, b_ref, m_count)
    qc = jnp.clip(q, 1, T)
    ws = (qc - 1) * R + (R - HALO)
    ywin = yext_ref[pl.ds(ws, WIN), :]
    mwin = maskext_ref[pl.ds(ws, WIN), :]
    z = (jnp.maximum(ywin * sc + sh, 0.0) * mwin).astype(jnp.bfloat16)
    acc = jnp.zeros((R, C), jnp.float32)
    for i in range(len(offs) // 2):
        o1, o2 = offs[2 * i], offs[2 * i + 1]
        a = jnp.concatenate(
            [z[o1 + HALO:o1 + HALO + R, :],
             z[o2 + HALO:o2 + HALO + R, :]], axis=1)
        acc = acc + jnp.dot(a, w_ref[i],
                            preferred_element_type=jnp.float32)
    if len(offs) % 2:
        o = offs[-1]
        acc = acc + jnp.dot(z[o + HALO:o + HALO + R, :], wl_ref[...],
                            preferred_element_type=jnp.float32)
    active = jnp.logical_and(q >= 1, q <= T)
    y = jnp.where(active, acc + bias_ref[...], 0.0)
    o_ref[...] = y.astype(o_ref.dtype)
    ym = y * maskext_ref[pl.ds(q * R, R), :]
    s_ref[0, 0:1, :] = jnp.sum(ym, axis=0, keepdims=True)
    s_ref[0, 1:2, :] = jnp.sum(ym * ym, axis=0, keepdims=True)


def _final_body(o1_ref, y_ref, st_ref, g_ref, b_ref, o_ref, *, m_count):
    sc, sh = _scale_shift(st_ref, g_ref, b_ref, m_count)
    o_ref[...] = jnp.maximum(o1_ref[...] + y_ref[...] * sc + sh, 0.0)


def kernel(x, down_w, down_b, bn_g, bn_b, conv0_w, conv0_b, bn0_g, bn0_b,
           conv1_w, conv1_b, bn1_g, bn1_b):
    N, Cin, D, H, W = x.shape
    Cout = down_w.shape[0]
    Do, Ho, Wo = D // 2, H // 2, W // 2
    Dp, Hp, Wp = Do + 2, Ho + 2, Wo + 2
    rows_n = Dp * Hp * Wp
    ROWS = N * rows_n
    RB = 2 * Hp * Wp
    R = 2 * RB
    T = ROWS // R
    TB = ROWS // RB
    EXT = ROWS + 2 * R
    PH_ROWS = RB + ROWS
    HALO = _round_up(Hp * Wp + Wp + 1, 8)
    M = float(N * Do * Ho * Wo)

    f32, bf16 = jnp.float32, jnp.bfloat16

    xw = jnp.transpose(x, (0, 2, 3, 4, 1)).reshape(N, D, H, Wo, 2 * Cin)

    rg = jnp.arange(EXT) - R
    rem = jnp.where(rg >= 0, rg, 0) % rows_n
    d = rem // (Hp * Wp)
    h = (rem % (Hp * Wp)) // Wp
    w_ = rem % Wp
    ok = ((rg >= 0) & (rg < ROWS)
          & (d >= 1) & (d < Dp - 1) & (h >= 1) & (h < Hp - 1)
          & (w_ >= 1) & (w_ < Wp - 1))
    mask = jnp.broadcast_to(ok.astype(bf16)[:, None], (EXT, Cout))

    wt = jnp.transpose(down_w, (1, 2, 3, 4, 0))
    zblk = jnp.zeros((Cin, Cout), f32)
    blocks, taps = [], []
    for kd in range(3):
        pd = 0 if kd == 1 else 1
        sd = -1 if kd == 0 else 0
        for kh in range(3):
            phh = 0 if kh == 1 else 1
            sh_ = -1 if kh == 0 else 0
            for g in (0, 1):
                sw = -1 if g == 0 else 0
                off = sd * (Hp * Wp) + sh_ * Wp + sw
                if g == 0:
                    blk = jnp.concatenate([zblk, wt[:, kd, kh, 0, :]], axis=0)
                else:
                    blk = jnp.concatenate(
                        [wt[:, kd, kh, 1, :], wt[:, kd, kh, 2, :]], axis=0)
                blocks.append(blk)
                taps.append((pd * 2 + phh, off))
    w18 = jnp.stack(blocks).astype(bf16)
    w9 = jnp.concatenate([w18[0::2], w18[1::2]], axis=1)

    def conv_wp(w):
        w27 = jnp.transpose(w, (2, 3, 4, 1, 0)).reshape(27, Cout, Cout)
        wp = jnp.concatenate([w27[0:26:2], w27[1:26:2]], axis=1)
        return wp.astype(bf16), w27[26].astype(bf16)

    offs27 = [dd * (Hp * Wp) + hh * Wp + ww
              for dd in (-1, 0, 1) for hh in (-1, 0, 1) for ww in (-1, 0, 1)]

    nine = TB // N
    def ima(q):
        s = jnp.maximum(q - 1, 0)
        return (s // nine, jnp.clip(2 * (s % nine) - 1, 0, Do - 1), 0, 0, 0)
    def imb(q):
        s = jnp.maximum(q - 1, 0)
        return (s // nine, jnp.clip(2 * (s % nine), 0, Do - 1), 0, 0, 0)
    phases = pl.pallas_call(
        functools.partial(_repack_body, nine=nine),
        grid=(TB + 1,),
        in_specs=[
            pl.BlockSpec((1, 2, H, Wo, Cout), ima),
            pl.BlockSpec((1, 2, H, Wo, Cout), imb),
        ],
        out_specs=[pl.BlockSpec((RB, Cout), lambda q: (q, 0))] * 4,
        out_shape=[jax.ShapeDtypeStruct((PH_ROWS, Cout), bf16)] * 4,
        compiler_params=_compiler_params(),
    )(xw, xw)

    y1_ext, st = pl.pallas_call(
        functools.partial(_down_body, R=R, RB=RB, DH=HALO, taps=taps, T=T),
        grid=(T + 2,),
        in_specs=[_full((PH_ROWS, Cout))] * 4 + [
            _full(w9.shape),
            _full((1, Cout)),
            pl.BlockSpec((R, Cout), lambda q: (q, 0)),
        ],
        out_specs=[
            pl.BlockSpec((R, Cout), lambda q: (q, 0)),
            pl.BlockSpec((1, 8, Cout), lambda q: (q, 0, 0)),
        ],
        out_shape=[
            jax.ShapeDtypeStruct((EXT, Cout), bf16),
            jax.ShapeDtypeStruct((T + 2, 8, Cout), f32),
        ],
        compiler_params=_compiler_params(),
    )(*phases, w9, down_b.reshape(1, Cout), mask)

    def conv_call(y_ext, st, g, b, wpair, bias):
        wp, wl = wpair
        return pl.pallas_call(
            functools.partial(_conv_body, R=R, HALO=HALO, offs=offs27,
                              T=T, m_count=M),
            grid=(T + 2,),
            in_specs=[
                _full((EXT, Cout)),
                _full((T + 2, 8, Cout)),
                _full((1, Cout)),
                _full((1, Cout)),
                _full(wp.shape),
                _full(wl.shape),
                _full((1, Cout)),
                _full((EXT, Cout)),
            ],
            out_specs=[
                pl.BlockSpec((R, Cout), lambda q: (q, 0)),
                pl.BlockSpec((1, 8, Cout), lambda q: (q, 0, 0)),
            ],
            out_shape=[
                jax.ShapeDtypeStruct((EXT, Cout), bf16),
                jax.ShapeDtypeStruct((T + 2, 8, Cout), f32),
            ],
            compiler_params=_compiler_params(),
        )(y_ext, st, g.reshape(1, Cout), b.reshape(1, Cout), wp, wl,
          bias.reshape(1, Cout), mask)

    y2_ext, st0 = conv_call(y1_ext, st, bn_g, bn_b,
                            conv_wp(conv0_w), conv0_b)
    y3_ext, st1 = conv_call(y2_ext, st0, bn0_g, bn0_b,
                            conv_wp(conv1_w), conv1_b)

    out_ext = pl.pallas_call(
        functools.partial(_final_body, m_count=M),
        grid=(T,),
        in_specs=[
            pl.BlockSpec((R, Cout), lambda m: (m + 1, 0)),
            pl.BlockSpec((R, Cout), lambda m: (m + 1, 0)),
            _full((T + 2, 8, Cout)),
            _full((1, Cout)),
            _full((1, Cout)),
        ],
        out_specs=pl.BlockSpec((R, Cout), lambda m: (m + 1, 0)),
        out_shape=jax.ShapeDtypeStruct((EXT, Cout), f32),
        compiler_params=_compiler_params(),
    )(y1_ext, y3_ext, st1, bn1_g.reshape(1, Cout), bn1_b.reshape(1, Cout))

    out = out_ext[R:R + ROWS].reshape(N, Dp, Hp, Wp, Cout)
    out = out[:, 1:Dp - 1, 1:Hp - 1, 1:Wp - 1, :]
    return jnp.transpose(out, (0, 4, 1, 2, 3))
